# baseline (speedup 1.0000x reference)
import functools

import jax
import jax.numpy as jnp
from jax import lax
from jax.experimental import pallas as pl
from jax.experimental.pallas import tpu as pltpu

N_REAL = 5000
B = 1024
NB = 5
NP = NB * B
PROB_THR = 0.5
IOU_THR = 0.5


def _nms_body(dcol_ref, s_row_ref, s_col_ref, out_ref,
              rankr_s, d15_s, sd_col_s, sd_row_s, geo_s, keep_s):
    f32 = jnp.float32
    col_np = lax.broadcasted_iota(jnp.int32, (1, NP), 1)

    n_valid = jnp.sum((s_row_ref[...] > PROB_THR).astype(jnp.int32))
    nba = lax.div(n_valid + (B - 1), B)

    sd_row_s[...] = jnp.zeros((8, NP), f32)

    def rank_blk(a, rrow):
        s_blk = s_col_ref[pl.ds(a * B, B), :]
        i_glob = (a * B
                  + lax.broadcasted_iota(jnp.int32, (B, 1), 0))
        s_all = s_row_ref[...]
        gt = (s_blk > s_all)
        tie = (s_blk == s_all) & (i_glob < col_np)
        return rrow + jnp.sum((gt | tie).astype(jnp.int32), axis=0,
                              keepdims=True)

    rankr_s[...] = lax.fori_loop(0, NB, rank_blk,
                                 jnp.zeros((1, NP), jnp.int32),
                                 unroll=False)

    d = dcol_ref[...]
    d_hi = lax.convert_element_type(d, jnp.bfloat16)
    r1 = d - lax.convert_element_type(d_hi, f32)
    d_mid = lax.convert_element_type(r1, jnp.bfloat16)
    d_lo = lax.convert_element_type(
        r1 - lax.convert_element_type(d_mid, f32), jnp.bfloat16)
    d15_s[...] = jnp.concatenate([d_hi, d_mid, d_lo], axis=1)

    def perm_blk(k, _):
        rows = k * B + lax.broadcasted_iota(jnp.int32, (B, 1), 0)
        p_row = (rankr_s[...] == rows).astype(jnp.bfloat16)
        r = lax.dot_general(p_row, d15_s[...],
                            (((1,), (0,)), ((), ())),
                            preferred_element_type=f32)
        blk_col = (r[:, 0:5] + r[:, 5:10]) + r[:, 10:15]
        sd_col_s[pl.ds(k * B, B), :] = blk_col
        sd_row_s[0:5, pl.ds(k * B, B)] = jnp.transpose(blk_col)
        return 0

    lax.fori_loop(0, nba, perm_blk, 0, unroll=False)

    x0 = sd_row_s[0:1, :]
    y0 = sd_row_s[1:2, :]
    x1 = sd_row_s[2:3, :]
    y1 = sd_row_s[3:4, :]
    lox = jnp.minimum(x0, x1)
    loy = jnp.minimum(y0, y1)
    hix = jnp.maximum(x0, x1)
    hiy = jnp.maximum(y0, y1)
    geo_s[0:1, :] = lox
    geo_s[1:2, :] = loy
    geo_s[2:3, :] = hix
    geo_s[3:4, :] = hiy
    geo_s[4:5, :] = (hix - lox) * (hiy - loy)

    keep_s[...] = (sd_row_s[4:5, :] > PROB_THR).astype(f32)

    ut_mask = (lax.broadcasted_iota(jnp.int32, (B, B), 0)
               < lax.broadcasted_iota(jnp.int32, (B, B), 1)).astype(f32)
    iota_b = lax.broadcasted_iota(jnp.int32, (1, B), 1)

    def nms_blk(k, _):
        cb = sd_col_s[pl.ds(k * B, B), :]
        bx0 = cb[:, 0:1]
        by0 = cb[:, 1:2]
        bx1 = cb[:, 2:3]
        by1 = cb[:, 3:4]
        lox_b = jnp.minimum(bx0, bx1)
        loy_b = jnp.minimum(by0, by1)
        hix_b = jnp.maximum(bx0, bx1)
        hiy_b = jnp.maximum(by0, by1)
        area_b = (hix_b - lox_b) * (hiy_b - loy_b)

        def iou_vs(lox_r, loy_r, hix_r, hiy_r, area_r):
            iw = jnp.clip(jnp.minimum(hix_b, hix_r)
                          - jnp.maximum(lox_b, lox_r), 0.0, None)
            ih = jnp.clip(jnp.minimum(hiy_b, hiy_r)
                          - jnp.maximum(loy_b, loy_r), 0.0, None)
            inter = iw * ih
            union = area_b + area_r - inter
            return inter / (union + 1e-8)

        sl = pl.ds(k * B, B)
        s_bb = (iou_vs(geo_s[0:1, sl], geo_s[1:2, sl],
                       geo_s[2:3, sl], geo_s[3:4, sl],
                       geo_s[4:5, sl]) > IOU_THR).astype(f32)
        s_ut = s_bb * ut_mask

        valid = keep_s[0:1, sl]

        def fix_cond(c):
            return c[1] > 0

        def fix_body(c):
            kb, _ = c
            supp = lax.dot_general(kb, s_ut, (((1,), (0,)), ((), ())),
                                   preferred_element_type=f32)
            new = valid * (supp < 0.5).astype(f32)
            changed = jnp.sum((new != kb).astype(jnp.int32))
            return (new, changed)

        keep_blk, _ = lax.while_loop(fix_cond, fix_body,
                                     (valid, jnp.int32(1)))
        keep_s[0:1, sl] = keep_blk

        s_all = (iou_vs(geo_s[0:1, :], geo_s[1:2, :],
                        geo_s[2:3, :], geo_s[3:4, :],
                        geo_s[4:5, :]) > IOU_THR).astype(f32)
        supp_all = lax.dot_general(keep_blk, s_all,
                                   (((1,), (0,)), ((), ())),
                                   preferred_element_type=f32)
        later = (col_np >= (k + 1) * B) & (supp_all > 0.5)
        keep_s[...] = keep_s[...] * (1.0 - later.astype(f32))
        return 0

    lax.fori_loop(0, nba, nms_blk, 0, unroll=False)

    m = keep_s[...]
    out_ref[0:5, :] = sd_row_s[0:5, :] * m
    out_ref[5:8, :] = jnp.zeros((3, NP), f32)


@jax.jit
def kernel(boxes, scores):
    pad = NP - N_REAL
    boxes_p = jnp.pad(boxes.astype(jnp.float32), ((0, pad), (0, 0)))
    scores_p = jnp.pad(scores.astype(jnp.float32), (0, pad),
                       constant_values=-1.0)
    s_row = scores_p.reshape(1, NP)
    s_col = scores_p.reshape(NP, 1)
    d_col = jnp.concatenate([boxes_p, s_col], axis=1)

    out_row = pl.pallas_call(
        _nms_body,
        out_shape=jax.ShapeDtypeStruct((8, NP), jnp.float32),
        scratch_shapes=[
            pltpu.VMEM((1, NP), jnp.int32),
            pltpu.VMEM((NP, 15), jnp.bfloat16),
            pltpu.VMEM((NP, 5), jnp.float32),
            pltpu.VMEM((8, NP), jnp.float32),
            pltpu.VMEM((8, NP), jnp.float32),
            pltpu.VMEM((1, NP), jnp.float32),
        ],
    )(d_col, s_row, s_col)

    return out_row[0:5, :N_REAL].T


# speedup vs baseline: 1.4031x; 1.4031x over previous
import functools

import jax
import jax.numpy as jnp
from jax import lax
from jax.experimental import pallas as pl
from jax.experimental.pallas import tpu as pltpu

N_REAL = 5000
B = 512
NB = 10
NP = NB * B
PROB_THR = 0.5
IOU_THR = 0.5


def _nms_body(dcol_ref, s_row_ref, s_col_ref, out_ref,
              rankr_s, d15_s, sd_col_s, sd_row_s, geo_s, keep_s):
    f32 = jnp.float32
    col_np = lax.broadcasted_iota(jnp.int32, (1, NP), 1)

    n_valid = jnp.sum((s_row_ref[...] > PROB_THR).astype(jnp.int32))
    nba = lax.div(n_valid + (B - 1), B)

    sd_row_s[...] = jnp.zeros((8, NP), f32)

    def rank_blk(a, rrow):
        s_blk = s_col_ref[pl.ds(a * B, B), :]
        i_glob = (a * B
                  + lax.broadcasted_iota(jnp.int32, (B, 1), 0))
        s_all = s_row_ref[...]
        gt = (s_blk > s_all)
        tie = (s_blk == s_all) & (i_glob < col_np)
        return rrow + jnp.sum((gt | tie).astype(jnp.int32), axis=0,
                              keepdims=True)

    rankr_s[...] = lax.fori_loop(0, NB, rank_blk,
                                 jnp.zeros((1, NP), jnp.int32),
                                 unroll=False)

    d = dcol_ref[...]
    d_hi = lax.convert_element_type(d, jnp.bfloat16)
    r1 = d - lax.convert_element_type(d_hi, f32)
    d_mid = lax.convert_element_type(r1, jnp.bfloat16)
    d_lo = lax.convert_element_type(
        r1 - lax.convert_element_type(d_mid, f32), jnp.bfloat16)
    d15_s[...] = jnp.concatenate([d_hi, d_mid, d_lo], axis=1)

    def perm_blk(k, _):
        rows = k * B + lax.broadcasted_iota(jnp.int32, (B, 1), 0)
        p_row = (rankr_s[...] == rows).astype(jnp.bfloat16)
        r = lax.dot_general(p_row, d15_s[...],
                            (((1,), (0,)), ((), ())),
                            preferred_element_type=f32)
        blk_col = (r[:, 0:5] + r[:, 5:10]) + r[:, 10:15]
        sd_col_s[pl.ds(k * B, B), :] = blk_col
        sd_row_s[0:5, pl.ds(k * B, B)] = jnp.transpose(blk_col)
        return 0

    lax.fori_loop(0, nba, perm_blk, 0, unroll=False)

    x0 = sd_row_s[0:1, :]
    y0 = sd_row_s[1:2, :]
    x1 = sd_row_s[2:3, :]
    y1 = sd_row_s[3:4, :]
    lox = jnp.minimum(x0, x1)
    loy = jnp.minimum(y0, y1)
    hix = jnp.maximum(x0, x1)
    hiy = jnp.maximum(y0, y1)
    geo_s[0:1, :] = lox
    geo_s[1:2, :] = loy
    geo_s[2:3, :] = hix
    geo_s[3:4, :] = hiy
    geo_s[4:5, :] = (hix - lox) * (hiy - loy)

    keep_s[...] = (sd_row_s[4:5, :] > PROB_THR).astype(f32)

    ut_mask = (lax.broadcasted_iota(jnp.int32, (B, B), 0)
               < lax.broadcasted_iota(jnp.int32, (B, B), 1)).astype(f32)
    iota_b = lax.broadcasted_iota(jnp.int32, (1, B), 1)

    def nms_blk(k, _):
        cb = sd_col_s[pl.ds(k * B, B), :]
        bx0 = cb[:, 0:1]
        by0 = cb[:, 1:2]
        bx1 = cb[:, 2:3]
        by1 = cb[:, 3:4]
        lox_b = jnp.minimum(bx0, bx1)
        loy_b = jnp.minimum(by0, by1)
        hix_b = jnp.maximum(bx0, bx1)
        hiy_b = jnp.maximum(by0, by1)
        area_b = (hix_b - lox_b) * (hiy_b - loy_b)

        def iou_vs(lox_r, loy_r, hix_r, hiy_r, area_r):
            iw = jnp.clip(jnp.minimum(hix_b, hix_r)
                          - jnp.maximum(lox_b, lox_r), 0.0, None)
            ih = jnp.clip(jnp.minimum(hiy_b, hiy_r)
                          - jnp.maximum(loy_b, loy_r), 0.0, None)
            inter = iw * ih
            union = area_b + area_r - inter
            return inter / (union + 1e-8)

        sl = pl.ds(k * B, B)
        s_bb = (iou_vs(geo_s[0:1, sl], geo_s[1:2, sl],
                       geo_s[2:3, sl], geo_s[3:4, sl],
                       geo_s[4:5, sl]) > IOU_THR).astype(f32)
        s_ut = s_bb * ut_mask

        valid = keep_s[0:1, sl]

        def fix_cond(c):
            return c[1] > 0

        def fix_body(c):
            kb, _ = c
            supp = lax.dot_general(kb, s_ut, (((1,), (0,)), ((), ())),
                                   preferred_element_type=f32)
            new = valid * (supp < 0.5).astype(f32)
            changed = jnp.sum((new != kb).astype(jnp.int32))
            return (new, changed)

        keep_blk, _ = lax.while_loop(fix_cond, fix_body,
                                     (valid, jnp.int32(1)))
        keep_s[0:1, sl] = keep_blk

        def supp_blk(j, _):
            sj = pl.ds(j * B, B)
            s_kj = (iou_vs(geo_s[0:1, sj], geo_s[1:2, sj],
                           geo_s[2:3, sj], geo_s[3:4, sj],
                           geo_s[4:5, sj]) > IOU_THR).astype(f32)
            supp = lax.dot_general(keep_blk, s_kj,
                                   (((1,), (0,)), ((), ())),
                                   preferred_element_type=f32)
            keep_s[0:1, sj] = keep_s[0:1, sj] * (supp < 0.5).astype(f32)
            return 0

        lax.fori_loop(k + 1, nba, supp_blk, 0, unroll=False)
        return 0

    lax.fori_loop(0, nba, nms_blk, 0, unroll=False)

    m = keep_s[...]
    out_ref[0:5, :] = sd_row_s[0:5, :] * m
    out_ref[5:8, :] = jnp.zeros((3, NP), f32)


@jax.jit
def kernel(boxes, scores):
    pad = NP - N_REAL
    boxes_p = jnp.pad(boxes.astype(jnp.float32), ((0, pad), (0, 0)))
    scores_p = jnp.pad(scores.astype(jnp.float32), (0, pad),
                       constant_values=-1.0)
    s_row = scores_p.reshape(1, NP)
    s_col = scores_p.reshape(NP, 1)
    d_col = jnp.concatenate([boxes_p, s_col], axis=1)

    out_row = pl.pallas_call(
        _nms_body,
        out_shape=jax.ShapeDtypeStruct((8, NP), jnp.float32),
        scratch_shapes=[
            pltpu.VMEM((1, NP), jnp.int32),
            pltpu.VMEM((NP, 15), jnp.bfloat16),
            pltpu.VMEM((NP, 5), jnp.float32),
            pltpu.VMEM((8, NP), jnp.float32),
            pltpu.VMEM((8, NP), jnp.float32),
            pltpu.VMEM((1, NP), jnp.float32),
        ],
    )(d_col, s_row, s_col)

    return out_row[0:5, :N_REAL].T


# speedup vs baseline: 1.7052x; 1.2153x over previous
import functools

import jax
import jax.numpy as jnp
from jax import lax
from jax.experimental import pallas as pl
from jax.experimental.pallas import tpu as pltpu

N_REAL = 5000
B = 512
NB = 10
NP = NB * B
PROB_THR = 0.5
IOU_THR = 0.5


def _nms_body(dcol_ref, s_row_ref, out_ref,
              rankr_s, d15_s, cd15_s, csc_col_s, cs_row_s,
              sd_col_s, sd_row_s, geo_s, keep_s):
    f32 = jnp.float32
    col_np = lax.broadcasted_iota(jnp.int32, (1, NP), 1)

    n_valid = jnp.sum((s_row_ref[...] > PROB_THR).astype(jnp.int32))
    nba = lax.div(n_valid + (B - 1), B)

    sd_row_s[...] = jnp.zeros((8, NP), f32)
    cs_row_s[...] = jnp.zeros((1, NP), f32)
    cd15_s[...] = jnp.zeros((NP, 15), jnp.bfloat16)

    def split3(x):
        hi = lax.convert_element_type(x, jnp.bfloat16)
        rem = x - lax.convert_element_type(hi, f32)
        mid = lax.convert_element_type(rem, jnp.bfloat16)
        lo = lax.convert_element_type(
            rem - lax.convert_element_type(mid, f32), jnp.bfloat16)
        return jnp.concatenate([hi, mid, lo], axis=1)

    d15_s[...] = split3(dcol_ref[...])

    valid_row = s_row_ref[...] > PROB_THR
    v_i32 = valid_row.astype(jnp.int32)
    cinc = v_i32
    sh = 1
    while sh < NP:
        cinc = cinc + jnp.concatenate(
            [jnp.zeros((1, sh), jnp.int32), cinc[:, :NP - sh]], axis=1)
        sh *= 2
    cexcl = cinc - v_i32
    dest = jnp.where(valid_row, cexcl,
                     n_valid + (col_np - cexcl))

    def comp_blk(k, _):
        rows = k * B + lax.broadcasted_iota(jnp.int32, (B, 1), 0)
        p_row = (dest == rows).astype(jnp.bfloat16)
        r = lax.dot_general(p_row, d15_s[...],
                            (((1,), (0,)), ((), ())),
                            preferred_element_type=f32)
        cb = (r[:, 0:5] + r[:, 5:10]) + r[:, 10:15]
        cd15_s[pl.ds(k * B, B), :] = split3(cb)
        csc_col_s[pl.ds(k * B, B), :] = cb[:, 4:5]
        cs_row_s[0:1, pl.ds(k * B, B)] = jnp.transpose(cb)[4:5, :]
        return 0

    lax.fori_loop(0, nba, comp_blk, 0, unroll=False)

    def rank_blk(a, rrow):
        s_blk = csc_col_s[pl.ds(a * B, B), :]
        i_glob = (a * B
                  + lax.broadcasted_iota(jnp.int32, (B, 1), 0))
        s_all = cs_row_s[...]
        gt = (s_blk > s_all)
        tie = (s_blk == s_all) & (i_glob < col_np)
        return rrow + jnp.sum((gt | tie).astype(jnp.int32), axis=0,
                              keepdims=True)

    rankr_s[...] = lax.fori_loop(0, nba, rank_blk,
                                 jnp.zeros((1, NP), jnp.int32),
                                 unroll=False)

    def perm_blk(k, _):
        rows = k * B + lax.broadcasted_iota(jnp.int32, (B, 1), 0)
        p_row = (rankr_s[...] == rows).astype(jnp.bfloat16)
        r = lax.dot_general(p_row, cd15_s[...],
                            (((1,), (0,)), ((), ())),
                            preferred_element_type=f32)
        blk_col = (r[:, 0:5] + r[:, 5:10]) + r[:, 10:15]
        sd_col_s[pl.ds(k * B, B), :] = blk_col
        sd_row_s[0:5, pl.ds(k * B, B)] = jnp.transpose(blk_col)
        return 0

    lax.fori_loop(0, nba, perm_blk, 0, unroll=False)

    x0 = sd_row_s[0:1, :]
    y0 = sd_row_s[1:2, :]
    x1 = sd_row_s[2:3, :]
    y1 = sd_row_s[3:4, :]
    lox = jnp.minimum(x0, x1)
    loy = jnp.minimum(y0, y1)
    hix = jnp.maximum(x0, x1)
    hiy = jnp.maximum(y0, y1)
    geo_s[0:1, :] = lox
    geo_s[1:2, :] = loy
    geo_s[2:3, :] = hix
    geo_s[3:4, :] = hiy
    geo_s[4:5, :] = (hix - lox) * (hiy - loy)

    keep_s[...] = ((sd_row_s[4:5, :] > PROB_THR)
                   & (col_np < n_valid)).astype(f32)

    ut_mask = (lax.broadcasted_iota(jnp.int32, (B, B), 0)
               < lax.broadcasted_iota(jnp.int32, (B, B), 1)).astype(f32)
    iota_b = lax.broadcasted_iota(jnp.int32, (1, B), 1)

    def nms_blk(k, _):
        cb = sd_col_s[pl.ds(k * B, B), :]
        bx0 = cb[:, 0:1]
        by0 = cb[:, 1:2]
        bx1 = cb[:, 2:3]
        by1 = cb[:, 3:4]
        lox_b = jnp.minimum(bx0, bx1)
        loy_b = jnp.minimum(by0, by1)
        hix_b = jnp.maximum(bx0, bx1)
        hiy_b = jnp.maximum(by0, by1)
        area_b = (hix_b - lox_b) * (hiy_b - loy_b)

        def iou_vs(lox_r, loy_r, hix_r, hiy_r, area_r):
            iw = jnp.clip(jnp.minimum(hix_b, hix_r)
                          - jnp.maximum(lox_b, lox_r), 0.0, None)
            ih = jnp.clip(jnp.minimum(hiy_b, hiy_r)
                          - jnp.maximum(loy_b, loy_r), 0.0, None)
            inter = iw * ih
            union = area_b + area_r - inter
            return inter / (union + 1e-8)

        sl = pl.ds(k * B, B)
        s_bb = (iou_vs(geo_s[0:1, sl], geo_s[1:2, sl],
                       geo_s[2:3, sl], geo_s[3:4, sl],
                       geo_s[4:5, sl]) > IOU_THR).astype(f32)
        s_ut = s_bb * ut_mask

        valid = keep_s[0:1, sl]

        def fix_cond(c):
            return c[1] > 0

        def fix_body(c):
            kb, _ = c
            supp = lax.dot_general(kb, s_ut, (((1,), (0,)), ((), ())),
                                   preferred_element_type=f32)
            new = valid * (supp < 0.5).astype(f32)
            changed = jnp.sum((new != kb).astype(jnp.int32))
            return (new, changed)

        keep_blk, _ = lax.while_loop(fix_cond, fix_body,
                                     (valid, jnp.int32(1)))
        keep_s[0:1, sl] = keep_blk

        def supp_blk(j, _):
            sj = pl.ds(j * B, B)
            s_kj = (iou_vs(geo_s[0:1, sj], geo_s[1:2, sj],
                           geo_s[2:3, sj], geo_s[3:4, sj],
                           geo_s[4:5, sj]) > IOU_THR).astype(f32)
            supp = lax.dot_general(keep_blk, s_kj,
                                   (((1,), (0,)), ((), ())),
                                   preferred_element_type=f32)
            keep_s[0:1, sj] = keep_s[0:1, sj] * (supp < 0.5).astype(f32)
            return 0

        lax.fori_loop(k + 1, nba, supp_blk, 0, unroll=False)
        return 0

    lax.fori_loop(0, nba, nms_blk, 0, unroll=False)

    m = keep_s[...]
    out_ref[0:5, :] = sd_row_s[0:5, :] * m
    out_ref[5:8, :] = jnp.zeros((3, NP), f32)


@jax.jit
def kernel(boxes, scores):
    pad = NP - N_REAL
    boxes_p = jnp.pad(boxes.astype(jnp.float32), ((0, pad), (0, 0)))
    scores_p = jnp.pad(scores.astype(jnp.float32), (0, pad),
                       constant_values=-1.0)
    s_row = scores_p.reshape(1, NP)
    s_col = scores_p.reshape(NP, 1)
    d_col = jnp.concatenate([boxes_p, s_col], axis=1)

    out_row = pl.pallas_call(
        _nms_body,
        out_shape=jax.ShapeDtypeStruct((8, NP), jnp.float32),
        scratch_shapes=[
            pltpu.VMEM((1, NP), jnp.int32),
            pltpu.VMEM((NP, 15), jnp.bfloat16),
            pltpu.VMEM((NP, 15), jnp.bfloat16),
            pltpu.VMEM((NP, 1), jnp.float32),
            pltpu.VMEM((1, NP), jnp.float32),
            pltpu.VMEM((NP, 5), jnp.float32),
            pltpu.VMEM((8, NP), jnp.float32),
            pltpu.VMEM((8, NP), jnp.float32),
            pltpu.VMEM((1, NP), jnp.float32),
        ],
    )(d_col, s_row)

    return out_row[0:5, :N_REAL].T
